# baseline (speedup 1.0000x reference)
import jax
import jax.numpy as jnp
from jax.experimental import pallas as pl

_HEADS = 4
_CH = 32
_HID = 128
_NEG = -1e30
_CLAMP = 60.0


def _dot(a, b):
    return jnp.dot(a, b, preferred_element_type=jnp.float32)


def _dot_t(a, b, dims):
    return jax.lax.dot_general(a, b, (dims, ((), ())),
                               preferred_element_type=jnp.float32)


def _fused_kernel(obs_ref, dx_ref, mask_ref, ff_ref,
                  poW1_ref, pob1_ref, pog_ref, pobe_ref, poW2_ref, pob2_ref,
                  pfW1_ref, pfb1_ref, pfg_ref, pfbe_ref, pfW2_ref, pfb2_ref,
                  gWsrc_ref, asrc_row_ref, onehotT_ref, wdalT_ref, eprod_ref,
                  gbias_ref, opW1t_ref, opW1b_ref, opb1_ref, opg_ref,
                  opbe_ref, opW2_ref, opb2_ref,
                  out_ref):
    n_rows = obs_ref.shape[0]
    n_feat = dx_ref.shape[1]
    inv_n = 1.0 / n_rows
    ones_n = jnp.ones((1, n_rows), dtype=jnp.float32)
    eye_f = jnp.eye(n_feat, dtype=jnp.float32)

    def bn_scale_shift(h, g, be):
        mu = _dot(ones_n, h) * inv_n
        var = _dot(ones_n, h * h) * inv_n - mu * mu
        s = jax.lax.rsqrt(var + 1e-5) * g
        return s, be - mu * s

    hf = _dot(ff_ref[...], pfW1_ref[...]) + pfb1_ref[...]
    muf = jnp.mean(hf, axis=0, keepdims=True)
    varf = jnp.mean(hf * hf, axis=0, keepdims=True) - muf * muf
    sf = jax.lax.rsqrt(varf + 1e-5) * pfg_ref[...]
    tf = pfbe_ref[...] - muf * sf
    feat_h = jax.nn.relu(_dot(hf * sf + tf, pfW2_ref[...]) + pfb2_ref[...])
    xs = _dot(feat_h, gWsrc_ref[...])
    al_s = _dot(xs * asrc_row_ref[...], onehotT_ref[...])
    cvec = _dot(eprod_ref[...], onehotT_ref[...])
    ones_col = jnp.ones((n_feat, 1), dtype=jnp.float32)
    xs_augs = [
        jnp.concatenate([xs[:, h * _CH:(h + 1) * _CH], ones_col], axis=1)
        for h in range(_HEADS)]

    h1 = _dot(obs_ref[...], poW1_ref[...]) + pob1_ref[...]
    s1, t1 = bn_scale_shift(h1, pog_ref[...], pobe_ref[...])
    obs_h = jax.nn.relu(_dot(h1 * s1 + t1, poW2_ref[...]) + pob2_ref[...])

    bias2 = _dot(gbias_ref[...], opW1b_ref[...]) + opb1_ref[...]

    n_chunks = 5
    rc = n_rows // n_chunks
    al_s_b = [jnp.broadcast_to(al_s[:, h:h + 1], (n_feat, rc))
              for h in range(_HEADS)]

    h2_parts = []
    for r in range(n_chunks):
        obs_c = jax.lax.slice(obs_h, (r * rc, 0), ((r + 1) * rc, _HID))
        al_dT = _dot_t(wdalT_ref[...], obs_c, ((1,), (1,)))
        dxT = _dot_t(eye_f, dx_ref[pl.ds(r * rc, rc), :], ((1,), (1,)))
        mvf = mask_ref[pl.ds(r * rc, rc), :].astype(jnp.float32)
        mnegT = _dot_t(eye_f, (mvf - 1.0) * -_NEG, ((1,), (1,)))
        gT_parts = []
        for h in range(_HEADS):
            raw = al_dT[h:h + 1, :] + (dxT * cvec[0, h] + al_s_b[h])
            alpha = jnp.maximum(raw, 0.2 * raw) + mnegT
            ex = jnp.exp(jnp.minimum(alpha, _CLAMP))
            res = _dot_t(xs_augs[h], ex, ((0,), (0,)))
            rec = 1.0 / (res[_CH:_CH + 1, :] + 1e-16)
            gT_parts.append(res[:_CH, :] * rec)
        gT = jnp.concatenate(gT_parts, axis=0)
        h2_parts.append(_dot(obs_c, opW1t_ref[...])
                        + _dot_t(gT, opW1b_ref[...], ((0,), (0,)))
                        + bias2)

    h2 = jnp.concatenate(h2_parts, axis=0)
    s2, t2 = bn_scale_shift(h2, opg_ref[...], opbe_ref[...])
    out_ref[...] = jax.nn.relu(_dot(h2 * s2 + t2, opW2_ref[...])
                               + opb2_ref[...])


def kernel(obs_features, feature_mask, feat_features, obs_adjs, data_x,
           poW1, pob1, pog, pobe, poW2, pob2,
           pfW1, pfb1, pfg, pfbe, pfW2, pfb2,
           gWsrc, gWdst, gWedge, gasrc, gadst, gaedge, gbias,
           opW1, opb1, opg, opbe, opW2, opb2):
    n_obs, n_feat = feature_mask.shape
    f32 = jnp.float32

    row = lambda v: v.reshape(1, -1).astype(f32)

    onehot = (jnp.arange(_HID, dtype=jnp.int32)[None, :] // _CH ==
              jnp.arange(_HEADS, dtype=jnp.int32)[:, None]).astype(f32)
    onehotT = onehot.T
    asrc_row = gasrc.reshape(1, _HID).astype(f32)
    adst_flat = gadst.reshape(1, _HID).astype(f32)
    wd_alT = (onehot * adst_flat) @ gWdst.astype(f32).T
    eprod = (gWedge.reshape(1, _HID) * gaedge.reshape(1, _HID)).astype(f32)

    out = pl.pallas_call(
        _fused_kernel,
        out_shape=jax.ShapeDtypeStruct((n_obs, _HID), f32),
    )(obs_features, data_x, feature_mask.astype(jnp.int8), feat_features,
      poW1, row(pob1), row(pog), row(pobe), poW2, row(pob2),
      pfW1, row(pfb1), row(pfg), row(pfbe), pfW2, row(pfb2),
      gWsrc, asrc_row, onehotT, wd_alT, eprod,
      row(gbias), opW1[:_HID], opW1[_HID:], row(opb1), row(opg),
      row(opbe), opW2, row(opb2))
    return out


# speedup vs baseline: 1.2622x; 1.2622x over previous
import jax
import jax.numpy as jnp
from jax.experimental import pallas as pl

_HEADS = 4
_CH = 32
_HID = 128
_NEG = -1e30
_CLAMP = 60.0


def _dot(a, b):
    return jnp.dot(a, b, preferred_element_type=jnp.float32)


def _dot_t(a, b, dims):
    return jax.lax.dot_general(a, b, (dims, ((), ())),
                               preferred_element_type=jnp.float32)


def _fused_kernel(obs_ref, dxT_ref, maskT_ref, ff_ref,
                  poW1_ref, pob1_ref, pog_ref, pobe_ref, poW2_ref, pob2_ref,
                  pfW1_ref, pfb1_ref, pfg_ref, pfbe_ref, pfW2_ref, pfb2_ref,
                  gWsrc_ref, asrc_row_ref, onehotT_ref, wdalT_ref, eprod_ref,
                  gbias_ref, opW1t_ref, opW1b_ref, opb1_ref, opg_ref,
                  opbe_ref, opW2_ref, opb2_ref,
                  out_ref):
    n_rows = obs_ref.shape[0]
    n_feat = dxT_ref.shape[0]

    def bn_scale_shift(h, g, be):
        mu = jnp.mean(h, axis=0, keepdims=True)
        var = jnp.mean(h * h, axis=0, keepdims=True) - mu * mu
        s = jax.lax.rsqrt(var + 1e-5) * g
        return s, be - mu * s

    hf = _dot(ff_ref[...], pfW1_ref[...]) + pfb1_ref[...]
    sf, tf = bn_scale_shift(hf, pfg_ref[...], pfbe_ref[...])
    feat_h = jax.nn.relu(_dot(hf * sf + tf, pfW2_ref[...]) + pfb2_ref[...])
    xs = _dot(feat_h, gWsrc_ref[...])
    al_s = _dot(xs * asrc_row_ref[...], onehotT_ref[...])
    cvec = _dot(eprod_ref[...], onehotT_ref[...])
    ones_col = jnp.ones((n_feat, 1), dtype=jnp.float32)
    xs_augs = [
        jnp.concatenate([xs[:, h * _CH:(h + 1) * _CH], ones_col], axis=1)
        for h in range(_HEADS)]

    h1 = _dot(obs_ref[...], poW1_ref[...]) + pob1_ref[...]
    s1, t1 = bn_scale_shift(h1, pog_ref[...], pobe_ref[...])
    obs_h = jax.nn.relu(_dot(h1 * s1 + t1, poW2_ref[...]) + pob2_ref[...])

    bias2 = _dot(gbias_ref[...], opW1b_ref[...]) + opb1_ref[...]

    n_chunks = 5
    rc = n_rows // n_chunks
    al_s_b = [jnp.broadcast_to(al_s[:, h:h + 1], (n_feat, rc))
              for h in range(_HEADS)]

    h2_parts = []
    for r in range(n_chunks):
        obs_c = jax.lax.slice(obs_h, (r * rc, 0), ((r + 1) * rc, _HID))
        al_dT = _dot_t(wdalT_ref[...], obs_c, ((1,), (1,)))
        dxT = dxT_ref[:, pl.ds(r * rc, rc)]
        mvT = maskT_ref[:, pl.ds(r * rc, rc)].astype(jnp.float32)
        mnegT = (mvT - 1.0) * -_NEG
        gT_parts = []
        for h in range(_HEADS):
            raw = al_dT[h:h + 1, :] + (dxT * cvec[0, h] + al_s_b[h])
            alpha = jnp.maximum(raw, 0.2 * raw) + mnegT
            ex = jnp.exp(jnp.minimum(alpha, _CLAMP))
            res = _dot_t(xs_augs[h], ex, ((0,), (0,)))
            rec = 1.0 / (res[_CH:_CH + 1, :] + 1e-16)
            gT_parts.append(res[:_CH, :] * rec)
        gT = jnp.concatenate(gT_parts, axis=0)
        h2_parts.append(_dot(obs_c, opW1t_ref[...])
                        + _dot_t(gT, opW1b_ref[...], ((0,), (0,)))
                        + bias2)

    h2 = jnp.concatenate(h2_parts, axis=0)
    s2, t2 = bn_scale_shift(h2, opg_ref[...], opbe_ref[...])
    out_ref[...] = jax.nn.relu(_dot(h2 * s2 + t2, opW2_ref[...])
                               + opb2_ref[...])


def kernel(obs_features, feature_mask, feat_features, obs_adjs, data_x,
           poW1, pob1, pog, pobe, poW2, pob2,
           pfW1, pfb1, pfg, pfbe, pfW2, pfb2,
           gWsrc, gWdst, gWedge, gasrc, gadst, gaedge, gbias,
           opW1, opb1, opg, opbe, opW2, opb2):
    n_obs, n_feat = feature_mask.shape
    f32 = jnp.float32

    row = lambda v: v.reshape(1, -1).astype(f32)

    onehot = (jnp.arange(_HID, dtype=jnp.int32)[None, :] // _CH ==
              jnp.arange(_HEADS, dtype=jnp.int32)[:, None]).astype(f32)
    onehotT = onehot.T
    asrc_row = gasrc.reshape(1, _HID).astype(f32)
    adst_flat = gadst.reshape(1, _HID).astype(f32)
    wd_alT = (onehot * adst_flat) @ gWdst.astype(f32).T
    eprod = (gWedge.reshape(1, _HID) * gaedge.reshape(1, _HID)).astype(f32)

    out = pl.pallas_call(
        _fused_kernel,
        out_shape=jax.ShapeDtypeStruct((n_obs, _HID), f32),
    )(obs_features, data_x.T, feature_mask.astype(jnp.int8).T, feat_features,
      poW1, row(pob1), row(pog), row(pobe), poW2, row(pob2),
      pfW1, row(pfb1), row(pfg), row(pfbe), pfW2, row(pfb2),
      gWsrc, asrc_row, onehotT, wd_alT, eprod,
      row(gbias), opW1[:_HID], opW1[_HID:], row(opb1), row(opg),
      row(opbe), opW2, row(opb2))
    return out
